# baseline (speedup 1.0000x reference)
import functools

import jax
import jax.numpy as jnp
from jax import lax
from jax.experimental import pallas as pl
from jax.experimental.pallas import tpu as pltpu
from jax.experimental.pallas import tpu_sc as plsc

NUM_EMB = 1000000
DIM = 32
BATCH = 16384
SEQ = 20
TOTAL = BATCH * SEQ

NW = 32
BPW = BATCH // NW
NTILE = BPW // 128
TIDX = 128 * SEQ
SCH = 5
NCH = SEQ // SCH
CLOOK = SCH * 128

NTC = NUM_EMB // 128
SLABS = NUM_EMB // 4


NBLK = NTC // 4


def _conv_body(wt_hbm, wtail_hbm, w2_hbm, s_v, d_v, sl0, sl1, ss0, ss1):
    c = lax.axis_index("c")
    s = lax.axis_index("s")
    wid = s * 2 + c
    lsem = [sl0, sl1]
    ssem = [ss0, ss1]
    iota16 = lax.iota(jnp.int32, 16)

    def blk_of(q):
        return wid + q * 32

    def valid(q):
        return blk_of(q) < NBLK

    def issue_load(q, par):
        pltpu.async_copy(wt_hbm.at[:, pl.ds(blk_of(q) * 512, 512)],
                         s_v.at[par], lsem[par])

    def transpose(par):
        src = s_v.at[par]
        dst = d_v.at[par]

        @plsc.parallel_loop(0, 1024, unroll=8)
        def _tr(k):
            tc = k >> 8
            jl = (k >> 3) & 31
            g = k & 7
            rowv = iota16 + ((g & 1) * 16)
            colv = jnp.full((16,), 0, jnp.int32) + (tc * 128 + jl * 4 + (g >> 1))
            v = plsc.load_gather(src, [rowv, colv])
            dst[tc * 32 + jl, pl.ds(g * 16, 16)] = v

    @pl.when(valid(0))
    def _prime():
        issue_load(0, 0)

    def do_cc(cc, carry):
        for par in range(2):
            q = cc * 2 + par
            nxt = 1 - par

            @pl.when(valid(q + 1))
            def _issue():
                issue_load(q + 1, nxt)

            @pl.when(valid(q))
            def _work():
                pltpu.make_async_copy(wt_hbm.at[:, pl.ds(0, 512)],
                                      s_v.at[par], lsem[par]).wait()

                @pl.when(valid(q) & (q >= 2))
                def _drain():
                    pltpu.make_async_copy(d_v.at[par],
                                          w2_hbm.at[pl.ds(0, 128), :],
                                          ssem[par]).wait()

                transpose(par)
                pltpu.async_copy(d_v.at[par],
                                 w2_hbm.at[pl.ds(blk_of(q) * 128, 128), :],
                                 ssem[par])
        return carry

    lax.fori_loop(0, 31, do_cc, 0)
    for par in range(2):
        pltpu.make_async_copy(d_v.at[par], w2_hbm.at[pl.ds(0, 128), :],
                              ssem[par]).wait()

    @pl.when(wid == 4)
    def _tail():
        pltpu.sync_copy(wtail_hbm, d_v.at[0, pl.ds(0, 16), :])
        pltpu.sync_copy(d_v.at[0, pl.ds(0, 16), :],
                        w2_hbm.at[pl.ds(NTC * 32, 16), :])


def _emb_body(ids_hbm, tab_hbm, out_hbm, idx_v, idxT_v, rows_v, stage_v,
              sg0, sg1, ss0, ss1):
    c = lax.axis_index("c")
    s = lax.axis_index("s")
    wid = s * 2 + c
    gsem = [sg0, sg1]
    ssem = [ss0, ss1]
    iota16 = lax.iota(jnp.int32, 16)
    iota_s = iota16 * SEQ
    c_blk0 = iota16 // 8
    c_blk1 = c_blk0 + 2
    c_ds = (iota16 % 8) * 128

    def do_tile(t, carry):
        base_b = wid * BPW + t * 128
        wb = wid * NTILE + t
        pltpu.sync_copy(ids_hbm.at[pl.ds(base_b * SEQ, TIDX)], idx_v)

        @plsc.parallel_loop(0, SEQ * 8, unroll=8)
        def _build(k):
            sq = k >> 3
            g = k & 7
            vec = plsc.load_gather(idx_v, [iota_s + (g * 16 * SEQ + sq)])
            idxT_v[pl.ds(sq * 128 + g * 16, 16)] = vec

        pltpu.async_copy(tab_hbm.at[idxT_v.at[pl.ds(0, CLOOK)]],
                         rows_v.at[0], gsem[0])

        def do_cc(cc, carry2):
            for par in range(2):
                q = cc * 2 + par
                nxt = 1 - par

                @pl.when(q < NCH - 1)
                def _issue():
                    pltpu.async_copy(
                        tab_hbm.at[idxT_v.at[pl.ds((q + 1) * CLOOK, CLOOK)]],
                        rows_v.at[nxt], gsem[nxt])

                pltpu.make_async_copy(tab_hbm.at[pl.ds(0, CLOOK)],
                                      rows_v.at[par], gsem[par]).wait()

                @pl.when(q >= 2)
                def _drain():
                    pltpu.make_async_copy(
                        stage_v.at[par],
                        out_hbm.at[pl.ds(0, SCH * 4), pl.ds(0, 1024)],
                        ssem[par]).wait()

                rows = rows_v.at[par]
                stg = stage_v.at[par]

                @plsc.parallel_loop(0, CLOOK, unroll=8)
                def _tr(r):
                    sl4 = (r >> 7) * 4
                    bl = r & 127
                    v0 = rows[r, pl.ds(0, 16)]
                    v1 = rows[r, pl.ds(16, 16)]
                    col = c_ds + bl
                    plsc.store_scatter(stg, [c_blk0 + sl4, col], v0)
                    plsc.store_scatter(stg, [c_blk1 + sl4, col], v1)

                pltpu.async_copy(
                    stg,
                    out_hbm.at[pl.ds(q * SCH * 4, SCH * 4),
                               pl.ds(wb * 1024, 1024)],
                    ssem[par])
            return carry2

        lax.fori_loop(0, NCH // 2, do_cc, 0)
        for par in range(2):
            pltpu.make_async_copy(stage_v.at[par],
                                  out_hbm.at[pl.ds(0, SCH * 4), pl.ds(0, 1024)],
                                  ssem[par]).wait()
        return carry

    lax.fori_loop(0, NTILE, do_tile, 0)


@jax.jit
def _run(token_ids, weight):
    mesh = plsc.VectorSubcoreMesh(core_axis_name="c", subcore_axis_name="s")
    conv = functools.partial(
        pl.kernel,
        mesh=mesh,
        out_type=jax.ShapeDtypeStruct((SLABS, 128), jnp.float32),
        scratch_types=[
            pltpu.VMEM((2, 32, 512), jnp.float32),
            pltpu.VMEM((2, 128, 128), jnp.float32),
            pltpu.SemaphoreType.DMA,
            pltpu.SemaphoreType.DMA,
            pltpu.SemaphoreType.DMA,
            pltpu.SemaphoreType.DMA,
        ],
        compiler_params=pltpu.CompilerParams(use_tc_tiling_on_sc=True,
                                             needs_layout_passes=False),
    )(_conv_body)
    emb = functools.partial(
        pl.kernel,
        mesh=mesh,
        out_type=jax.ShapeDtypeStruct((SEQ * DIM // 8, BATCH * 8), jnp.float32),
        scratch_types=[
            pltpu.VMEM((TIDX,), jnp.int32),
            pltpu.VMEM((TIDX,), jnp.int32),
            pltpu.VMEM((2, CLOOK, DIM), jnp.float32),
            pltpu.VMEM((2, SCH * 4, 1024), jnp.float32),
            pltpu.SemaphoreType.DMA,
            pltpu.SemaphoreType.DMA,
            pltpu.SemaphoreType.DMA,
            pltpu.SemaphoreType.DMA,
        ],
        compiler_params=pltpu.CompilerParams(use_tc_tiling_on_sc=False,
                                             needs_layout_passes=False),
    )(_emb_body)

    wtail = weight[NTC * 128:].reshape(16, 128)
    w2 = conv(weight.T, wtail)
    w2r = w2.reshape(NUM_EMB, DIM)
    ids = token_ids.reshape(-1).astype(jnp.int32)
    out2 = emb(ids, w2r)
    out5 = out2.reshape(SEQ, DIM // 8, BATCH // 128, 8, 128)
    return out5.transpose(2, 4, 0, 1, 3).reshape(BATCH, SEQ, DIM)


def kernel(token_ids, weight):
    return _run(token_ids, weight)


# speedup vs baseline: 1.0005x; 1.0005x over previous
import functools

import jax
import jax.numpy as jnp
from jax import lax
from jax.experimental import pallas as pl
from jax.experimental.pallas import tpu as pltpu
from jax.experimental.pallas import tpu_sc as plsc

NUM_EMB = 1000000
DIM = 32
BATCH = 16384
SEQ = 20
TOTAL = BATCH * SEQ

NW = 32
BPW = BATCH // NW
NTILE = BPW // 128
TIDX = 128 * SEQ
SCH = 5
NCH = SEQ // SCH
CLOOK = SCH * 128

NTC = NUM_EMB // 128
SLABS = NUM_EMB // 4


NBLK = NTC // 4


def _conv_body(wt_hbm, wtail_hbm, w2_hbm, s_v, d_v, sl0, sl1, ss0, ss1):
    c = lax.axis_index("c")
    s = lax.axis_index("s")
    wid = s * 2 + c
    lsem = [sl0, sl1]
    ssem = [ss0, ss1]
    iota16 = lax.iota(jnp.int32, 16)

    def blk_of(q):
        return wid + q * 32

    def valid(q):
        return blk_of(q) < NBLK

    def issue_load(q, par):
        pltpu.async_copy(wt_hbm.at[:, pl.ds(blk_of(q) * 512, 512)],
                         s_v.at[par, :, pl.ds(0, 512)], lsem[par])

    def transpose(par):
        src = s_v.at[par]
        dst = d_v.at[par]

        @plsc.parallel_loop(0, 1024, unroll=8)
        def _tr(k):
            tc = k >> 8
            jl = (k >> 3) & 31
            g = k & 7
            rowv = iota16 + ((g & 1) * 16)
            colv = jnp.full((16,), 0, jnp.int32) + (tc * 128 + jl * 4 + (g >> 1))
            v = plsc.load_gather(src, [rowv, colv])
            dst[tc * 32 + jl, pl.ds(g * 16, 16)] = v

    @pl.when(valid(0))
    def _prime():
        issue_load(0, 0)

    def do_cc(cc, carry):
        for par in range(2):
            q = cc * 2 + par
            nxt = 1 - par

            @pl.when(valid(q + 1))
            def _issue():
                issue_load(q + 1, nxt)

            @pl.when(valid(q))
            def _work():
                pltpu.make_async_copy(wt_hbm.at[:, pl.ds(0, 512)],
                                      s_v.at[par, :, pl.ds(0, 512)],
                                      lsem[par]).wait()

                @pl.when(valid(q) & (q >= 2))
                def _drain():
                    pltpu.make_async_copy(d_v.at[par],
                                          w2_hbm.at[pl.ds(0, 128), :],
                                          ssem[par]).wait()

                transpose(par)
                pltpu.async_copy(d_v.at[par],
                                 w2_hbm.at[pl.ds(blk_of(q) * 128, 128), :],
                                 ssem[par])
        return carry

    lax.fori_loop(0, 31, do_cc, 0)
    for par in range(2):
        pltpu.make_async_copy(d_v.at[par], w2_hbm.at[pl.ds(0, 128), :],
                              ssem[par]).wait()

    @pl.when(wid == 4)
    def _tail():
        pltpu.sync_copy(wtail_hbm, d_v.at[0, pl.ds(0, 16), :])
        pltpu.sync_copy(d_v.at[0, pl.ds(0, 16), :],
                        w2_hbm.at[pl.ds(NTC * 32, 16), :])


def _emb_body(ids_hbm, tab_hbm, out_hbm, idx_v, idxT_v, rows_v, stage_v,
              sg0, sg1, ss0, ss1):
    c = lax.axis_index("c")
    s = lax.axis_index("s")
    wid = s * 2 + c
    gsem = [sg0, sg1]
    ssem = [ss0, ss1]
    iota16 = lax.iota(jnp.int32, 16)
    iota_s = iota16 * SEQ
    c_blk0 = iota16 // 8
    c_blk1 = c_blk0 + 2
    c_ds = (iota16 % 8) * 128

    def do_tile(t, carry):
        base_b = wid * BPW + t * 128
        wb = wid * NTILE + t
        pltpu.sync_copy(ids_hbm.at[pl.ds(base_b * SEQ, TIDX)], idx_v)

        @plsc.parallel_loop(0, SEQ * 8, unroll=8)
        def _build(k):
            sq = k >> 3
            g = k & 7
            vec = plsc.load_gather(idx_v, [iota_s + (g * 16 * SEQ + sq)])
            idxT_v[pl.ds(sq * 128 + g * 16, 16)] = vec

        pltpu.async_copy(tab_hbm.at[idxT_v.at[pl.ds(0, CLOOK)]],
                         rows_v.at[0], gsem[0])

        def do_cc(cc, carry2):
            for par in range(2):
                q = cc * 2 + par
                nxt = 1 - par

                @pl.when(q < NCH - 1)
                def _issue():
                    pltpu.async_copy(
                        tab_hbm.at[idxT_v.at[pl.ds((q + 1) * CLOOK, CLOOK)]],
                        rows_v.at[nxt], gsem[nxt])

                pltpu.make_async_copy(tab_hbm.at[pl.ds(0, CLOOK)],
                                      rows_v.at[par], gsem[par]).wait()

                @pl.when(q >= 2)
                def _drain():
                    pltpu.make_async_copy(
                        stage_v.at[par],
                        out_hbm.at[pl.ds(0, SCH * 4), pl.ds(0, 1024)],
                        ssem[par]).wait()

                rows = rows_v.at[par]
                stg = stage_v.at[par]

                @plsc.parallel_loop(0, CLOOK, unroll=8)
                def _tr(r):
                    sl4 = (r >> 7) * 4
                    bl = r & 127
                    v0 = rows[r, pl.ds(0, 16)]
                    v1 = rows[r, pl.ds(16, 16)]
                    col = c_ds + bl
                    plsc.store_scatter(stg, [c_blk0 + sl4, col], v0)
                    plsc.store_scatter(stg, [c_blk1 + sl4, col], v1)

                pltpu.async_copy(
                    stg,
                    out_hbm.at[pl.ds(q * SCH * 4, SCH * 4),
                               pl.ds(wb * 1024, 1024)],
                    ssem[par])
            return carry2

        lax.fori_loop(0, NCH // 2, do_cc, 0)
        for par in range(2):
            pltpu.make_async_copy(stage_v.at[par],
                                  out_hbm.at[pl.ds(0, SCH * 4), pl.ds(0, 1024)],
                                  ssem[par]).wait()
        return carry

    lax.fori_loop(0, NTILE, do_tile, 0)


@jax.jit
def _run(token_ids, weight):
    mesh = plsc.VectorSubcoreMesh(core_axis_name="c", subcore_axis_name="s")
    conv = functools.partial(
        pl.kernel,
        mesh=mesh,
        out_type=jax.ShapeDtypeStruct((SLABS, 128), jnp.float32),
        scratch_types=[
            pltpu.VMEM((2, 32, 513), jnp.float32),
            pltpu.VMEM((2, 128, 128), jnp.float32),
            pltpu.SemaphoreType.DMA,
            pltpu.SemaphoreType.DMA,
            pltpu.SemaphoreType.DMA,
            pltpu.SemaphoreType.DMA,
        ],
        compiler_params=pltpu.CompilerParams(use_tc_tiling_on_sc=True,
                                             needs_layout_passes=False),
    )(_conv_body)
    emb = functools.partial(
        pl.kernel,
        mesh=mesh,
        out_type=jax.ShapeDtypeStruct((SEQ * DIM // 8, BATCH * 8), jnp.float32),
        scratch_types=[
            pltpu.VMEM((TIDX,), jnp.int32),
            pltpu.VMEM((TIDX,), jnp.int32),
            pltpu.VMEM((2, CLOOK, DIM), jnp.float32),
            pltpu.VMEM((2, SCH * 4, 1024), jnp.float32),
            pltpu.SemaphoreType.DMA,
            pltpu.SemaphoreType.DMA,
            pltpu.SemaphoreType.DMA,
            pltpu.SemaphoreType.DMA,
        ],
        compiler_params=pltpu.CompilerParams(use_tc_tiling_on_sc=False,
                                             needs_layout_passes=False),
    )(_emb_body)

    wtail = weight[NTC * 128:].reshape(16, 128)
    w2 = conv(weight.T, wtail)
    w2r = w2.reshape(NUM_EMB, DIM)
    ids = token_ids.reshape(-1).astype(jnp.int32)
    out2 = emb(ids, w2r)
    out5 = out2.reshape(SEQ, DIM // 8, BATCH // 128, 8, 128)
    return out5.transpose(2, 4, 0, 1, 3).reshape(BATCH, SEQ, DIM)


def kernel(token_ids, weight):
    return _run(token_ids, weight)


# speedup vs baseline: 1.0561x; 1.0556x over previous
import functools

import jax
import jax.numpy as jnp
from jax import lax
from jax.experimental import pallas as pl
from jax.experimental.pallas import tpu as pltpu
from jax.experimental.pallas import tpu_sc as plsc

NUM_EMB = 1000000
DIM = 32
BATCH = 16384
SEQ = 20
TOTAL = BATCH * SEQ

NW = 32
BPW = BATCH // NW
NTILE = BPW // 128
TIDX = 128 * SEQ
SCH = 5
NCH = SEQ // SCH
CLOOK = SCH * 128

NTC = NUM_EMB // 128
SLABS = NUM_EMB // 4


def _conv_body(wt_hbm, wtail_hbm, w2_hbm, s_v, d_v, sl0, sl1, ss0, ss1):
    c = lax.axis_index("c")
    s = lax.axis_index("s")
    wid = s * 2 + c
    lsem = [sl0, sl1]
    ssem = [ss0, ss1]
    iota16 = lax.iota(jnp.int32, 16)

    def tc_of(q):
        return wid + q * 32

    def issue_load(q, par):
        pltpu.async_copy(wt_hbm.at[:, pl.ds(tc_of(q) * 128, 128)],
                         s_v.at[par], lsem[par])

    def transpose(par):
        src = s_v.at[par]
        dst = d_v.at[par]

        @plsc.parallel_loop(0, 256, unroll=8)
        def _tr(k):
            jl = k >> 3
            g = k & 7
            rowv = iota16 + ((g & 1) * 16)
            colv = jnp.full((16,), 0, jnp.int32) + (jl * 4 + (g >> 1))
            v = plsc.load_gather(src, [rowv, colv])
            dst[jl, pl.ds(g * 16, 16)] = v

    issue_load(0, 0)

    def do_cc(cc, carry):
        for par in range(2):
            q = cc * 2 + par
            nxt = 1 - par

            @pl.when(q + 1 < 244)
            def _issue():
                issue_load(q + 1, nxt)

            pltpu.make_async_copy(wt_hbm.at[:, pl.ds(0, 128)],
                                  s_v.at[par], lsem[par]).wait()

            @pl.when(q >= 2)
            def _drain():
                pltpu.make_async_copy(d_v.at[par], w2_hbm.at[pl.ds(0, 32), :],
                                      ssem[par]).wait()

            transpose(par)
            pltpu.async_copy(d_v.at[par],
                             w2_hbm.at[pl.ds(tc_of(q) * 32, 32), :],
                             ssem[par])
        return carry

    lax.fori_loop(0, 122, do_cc, 0)
    for par in range(2):
        pltpu.make_async_copy(d_v.at[par], w2_hbm.at[pl.ds(0, 32), :],
                              ssem[par]).wait()

    @pl.when(wid <= 3)
    def _last_full():
        pltpu.sync_copy(wt_hbm.at[:, pl.ds(tc_of(244) * 128, 128)], s_v.at[0])
        transpose(0)
        pltpu.sync_copy(d_v.at[0], w2_hbm.at[pl.ds(tc_of(244) * 32, 32), :])

    @pl.when(wid == 4)
    def _tail():
        pltpu.sync_copy(wtail_hbm, d_v.at[0, pl.ds(0, 16), :])
        pltpu.sync_copy(d_v.at[0, pl.ds(0, 16), :],
                        w2_hbm.at[pl.ds(NTC * 32, 16), :])


def _emb_body(ids_hbm, tab_hbm, out_hbm, idx_v, idxT_v, rows_v, stage_v,
              sg0, sg1, ss0, ss1):
    c = lax.axis_index("c")
    s = lax.axis_index("s")
    wid = s * 2 + c
    gsem = [sg0, sg1]
    ssem = [ss0, ss1]
    iota16 = lax.iota(jnp.int32, 16)
    iota_s = iota16 * SEQ
    c_blk0 = iota16 // 8
    c_blk1 = c_blk0 + 2
    c_ds = (iota16 % 8) * 128

    def do_tile(t, carry):
        base_b = wid * BPW + t * 128
        wb = wid * NTILE + t
        pltpu.sync_copy(ids_hbm.at[pl.ds(base_b * SEQ, TIDX)], idx_v)

        @plsc.parallel_loop(0, SEQ * 8, unroll=8)
        def _build(k):
            sq = k >> 3
            g = k & 7
            vec = plsc.load_gather(idx_v, [iota_s + (g * 16 * SEQ + sq)])
            idxT_v[pl.ds(sq * 128 + g * 16, 16)] = vec

        pltpu.async_copy(tab_hbm.at[idxT_v.at[pl.ds(0, CLOOK)]],
                         rows_v.at[0], gsem[0])

        def do_cc(cc, carry2):
            for par in range(2):
                q = cc * 2 + par
                nxt = 1 - par

                @pl.when(q < NCH - 1)
                def _issue():
                    pltpu.async_copy(
                        tab_hbm.at[idxT_v.at[pl.ds((q + 1) * CLOOK, CLOOK)]],
                        rows_v.at[nxt], gsem[nxt])

                pltpu.make_async_copy(tab_hbm.at[pl.ds(0, CLOOK)],
                                      rows_v.at[par], gsem[par]).wait()

                @pl.when(q >= 2)
                def _drain():
                    pltpu.make_async_copy(
                        stage_v.at[par],
                        out_hbm.at[pl.ds(0, SCH * 4), pl.ds(0, 1024)],
                        ssem[par]).wait()

                rows = rows_v.at[par]
                stg = stage_v.at[par]

                @plsc.parallel_loop(0, CLOOK, unroll=8)
                def _tr(r):
                    sl4 = (r >> 7) * 4
                    bl = r & 127
                    v0 = rows[r, pl.ds(0, 16)]
                    v1 = rows[r, pl.ds(16, 16)]
                    col = c_ds + bl
                    plsc.store_scatter(stg, [c_blk0 + sl4, col], v0)
                    plsc.store_scatter(stg, [c_blk1 + sl4, col], v1)

                pltpu.async_copy(
                    stg,
                    out_hbm.at[pl.ds(q * SCH * 4, SCH * 4),
                               pl.ds(wb * 1024, 1024)],
                    ssem[par])
            return carry2

        lax.fori_loop(0, NCH // 2, do_cc, 0)
        for par in range(2):
            pltpu.make_async_copy(stage_v.at[par],
                                  out_hbm.at[pl.ds(0, SCH * 4), pl.ds(0, 1024)],
                                  ssem[par]).wait()
        return carry

    lax.fori_loop(0, NTILE, do_tile, 0)


@jax.jit
def _run(token_ids, weight):
    mesh = plsc.VectorSubcoreMesh(core_axis_name="c", subcore_axis_name="s")
    conv = functools.partial(
        pl.kernel,
        mesh=mesh,
        out_type=jax.ShapeDtypeStruct((SLABS, 128), jnp.float32),
        scratch_types=[
            pltpu.VMEM((2, 32, 128), jnp.float32),
            pltpu.VMEM((2, 32, 128), jnp.float32),
            pltpu.SemaphoreType.DMA,
            pltpu.SemaphoreType.DMA,
            pltpu.SemaphoreType.DMA,
            pltpu.SemaphoreType.DMA,
        ],
        compiler_params=pltpu.CompilerParams(use_tc_tiling_on_sc=True,
                                             needs_layout_passes=False),
    )(_conv_body)
    emb = functools.partial(
        pl.kernel,
        mesh=mesh,
        out_type=jax.ShapeDtypeStruct((SEQ * DIM // 8, BATCH * 8), jnp.float32),
        scratch_types=[
            pltpu.VMEM((TIDX,), jnp.int32),
            pltpu.VMEM((TIDX,), jnp.int32),
            pltpu.VMEM((2, CLOOK, DIM), jnp.float32),
            pltpu.VMEM((2, SCH * 4, 1024), jnp.float32),
            pltpu.SemaphoreType.DMA,
            pltpu.SemaphoreType.DMA,
            pltpu.SemaphoreType.DMA,
            pltpu.SemaphoreType.DMA,
        ],
        compiler_params=pltpu.CompilerParams(use_tc_tiling_on_sc=False,
                                             needs_layout_passes=False),
    )(_emb_body)

    wtail = weight[NTC * 128:].reshape(16, 128)
    w2 = conv(weight.T, wtail)
    w2r = w2.reshape(NUM_EMB, DIM)
    ids = token_ids.reshape(-1).astype(jnp.int32)
    out2 = emb(ids, w2r)
    out5 = out2.reshape(SEQ, DIM // 8, BATCH // 128, 8, 128)
    return out5.transpose(2, 4, 0, 1, 3).reshape(BATCH, SEQ, DIM)


def kernel(token_ids, weight):
    return _run(token_ids, weight)
